# Initial kernel scaffold; baseline (speedup 1.0000x reference)
import jax
import jax.numpy as jnp
from jax.experimental import pallas as pl


def kernel(x, edge_index, W1, b1, W2, b2, Wd, bd):
    raise NotImplementedError("write your pallas kernel here")


# speedup vs baseline: 15.1840x; 15.1840x over previous
import functools

import jax
import jax.numpy as jnp
from jax import lax
from jax.experimental import pallas as pl
from jax.experimental.pallas import tpu as pltpu
from jax.experimental.pallas import tpu_sc as plsc

_N = 10000
_E = 320000
_DIN = 128
_DH = 64

_NC = 2
_NS = 16
_NW = _NC * _NS
_C = 128
_NCH = 80
_EPAD = _NW * _NCH * _C
_NP = 10240
_RPS = _NP // _NS

_MESH = dict(core_axis_name="c", subcore_axis_name="s", num_cores=_NC,
             num_subcores=_NS)


@functools.partial(
    pl.kernel,
    out_type=jax.ShapeDtypeStruct((_NC, _NP, 8), jnp.float32),
    mesh=plsc.VectorSubcoreMesh(**_MESH),
    scratch_types=[
        pltpu.VMEM((_NCH, _C), jnp.int32),
        pltpu.VMEM((_C, 8), jnp.float32),
        pltpu.VMEM_SHARED((_NP, 8), jnp.float32),
    ],
    compiler_params=pltpu.CompilerParams(use_tc_tiling_on_sc=False),
)
def _deg(dst_hbm, ones_hbm, zeros_hbm, out_hbm, didx, ones_v, acc):
    c = lax.axis_index("c")
    s = lax.axis_index("s")
    wid = c * _NS + s
    pltpu.sync_copy(dst_hbm.at[wid], didx)
    pltpu.sync_copy(ones_hbm, ones_v)
    pltpu.sync_copy(zeros_hbm, acc.at[pl.ds(s * _RPS, _RPS)])
    plsc.subcore_barrier()

    def body(j, carry):
        pltpu.sync_copy(ones_v, acc.at[didx.at[j]], add=True)
        return carry

    lax.fori_loop(0, _NCH, body, 0)
    plsc.subcore_barrier()
    pltpu.sync_copy(acc.at[pl.ds(s * _RPS, _RPS)],
                    out_hbm.at[c, pl.ds(s * _RPS, _RPS)])


@functools.partial(
    pl.kernel,
    out_type=jax.ShapeDtypeStruct((_NC, _NP, _DH), jnp.float32),
    mesh=plsc.VectorSubcoreMesh(**_MESH),
    scratch_types=[
        pltpu.VMEM((_NCH, _C), jnp.int32),
        pltpu.VMEM((_NCH, _C), jnp.int32),
        pltpu.VMEM((_C, _DH), jnp.float32),
        pltpu.VMEM((_C, _DH), jnp.float32),
        pltpu.SemaphoreType.DMA,
        pltpu.SemaphoreType.DMA,
        pltpu.VMEM_SHARED((_NP, _DH), jnp.float32),
    ],
    compiler_params=pltpu.CompilerParams(use_tc_tiling_on_sc=False),
)
def _agg(g_hbm, src_hbm, dst_hbm, zeros_hbm, out_hbm,
         sidx, didx, rows0, rows1, sem0, sem1, acc):
    c = lax.axis_index("c")
    s = lax.axis_index("s")
    wid = c * _NS + s
    pltpu.sync_copy(src_hbm.at[wid], sidx)
    pltpu.sync_copy(dst_hbm.at[wid], didx)
    pltpu.sync_copy(zeros_hbm, acc.at[pl.ds(s * _RPS, _RPS)])
    plsc.subcore_barrier()

    pltpu.async_copy(g_hbm.at[sidx.at[0]], rows0, sem0)

    def body(i, carry):
        j = 2 * i
        pltpu.make_async_copy(g_hbm.at[sidx.at[j]], rows0, sem0).wait()
        pltpu.async_copy(g_hbm.at[sidx.at[j + 1]], rows1, sem1)
        pltpu.sync_copy(rows0, acc.at[didx.at[j]], add=True)
        pltpu.make_async_copy(g_hbm.at[sidx.at[j + 1]], rows1, sem1).wait()
        jn = jnp.minimum(j + 2, _NCH - 1)
        pltpu.async_copy(g_hbm.at[sidx.at[jn]], rows0, sem0)
        pltpu.sync_copy(rows1, acc.at[didx.at[j + 1]], add=True)
        return carry

    lax.fori_loop(0, _NCH // 2, body, 0)
    pltpu.make_async_copy(g_hbm.at[sidx.at[_NCH - 1]], rows0, sem0).wait()
    plsc.subcore_barrier()
    pltpu.sync_copy(acc.at[pl.ds(s * _RPS, _RPS)],
                    out_hbm.at[c, pl.ds(s * _RPS, _RPS)])


_BLK = 1024
_G = _NP // _BLK


def _dis_block(degp_ref):
    deg = degp_ref[0, :, 0:1] + degp_ref[1, :, 0:1] + 1.0
    return lax.rsqrt(deg)


def _mm1_body(x_ref, w_ref, degp_ref, out_ref):
    dis = _dis_block(degp_ref)
    h = jnp.dot(x_ref[...], w_ref[...], preferred_element_type=jnp.float32)
    out_ref[...] = h * dis


def _mm2_body(sp_ref, g_ref, degp_ref, b_ref, w_ref, out_ref):
    dis = _dis_block(degp_ref)
    ssum = sp_ref[0] + sp_ref[1] + g_ref[...]
    h = jnp.maximum(ssum * dis + b_ref[...], 0.0)
    out_ref[...] = jnp.dot(h, w_ref[...],
                           preferred_element_type=jnp.float32) * dis


def _mm3_body(sp_ref, g_ref, degp_ref, b_ref, wd_ref, bd_ref,
              h_ref, dec_ref):
    dis = _dis_block(degp_ref)
    ssum = sp_ref[0] + sp_ref[1] + g_ref[...]
    h = jnp.maximum(ssum * dis + b_ref[...], 0.0)
    h_ref[...] = h
    dec_ref[...] = jnp.dot(h, wd_ref[...],
                           preferred_element_type=jnp.float32) + bd_ref[...]


def _mm1(xp, W1, degp):
    return pl.pallas_call(
        _mm1_body,
        grid=(_G,),
        in_specs=[
            pl.BlockSpec((_BLK, _DIN), lambda i: (i, 0)),
            pl.BlockSpec((_DIN, _DH), lambda i: (0, 0)),
            pl.BlockSpec((_NC, _BLK, 8), lambda i: (0, i, 0)),
        ],
        out_specs=pl.BlockSpec((_BLK, _DH), lambda i: (i, 0)),
        out_shape=jax.ShapeDtypeStruct((_NP, _DH), jnp.float32),
    )(xp, W1, degp)


def _mm2(sp, g1, degp, b1, W2):
    return pl.pallas_call(
        _mm2_body,
        grid=(_G,),
        in_specs=[
            pl.BlockSpec((_NC, _BLK, _DH), lambda i: (0, i, 0)),
            pl.BlockSpec((_BLK, _DH), lambda i: (i, 0)),
            pl.BlockSpec((_NC, _BLK, 8), lambda i: (0, i, 0)),
            pl.BlockSpec((1, _DH), lambda i: (0, 0)),
            pl.BlockSpec((_DH, _DH), lambda i: (0, 0)),
        ],
        out_specs=pl.BlockSpec((_BLK, _DH), lambda i: (i, 0)),
        out_shape=jax.ShapeDtypeStruct((_NP, _DH), jnp.float32),
    )(sp, g1, degp, b1, W2)


def _mm3(sp, g2, degp, b2, Wd, bd):
    return pl.pallas_call(
        _mm3_body,
        grid=(_G,),
        in_specs=[
            pl.BlockSpec((_NC, _BLK, _DH), lambda i: (0, i, 0)),
            pl.BlockSpec((_BLK, _DH), lambda i: (i, 0)),
            pl.BlockSpec((_NC, _BLK, 8), lambda i: (0, i, 0)),
            pl.BlockSpec((1, _DH), lambda i: (0, 0)),
            pl.BlockSpec((_DH, _DIN), lambda i: (0, 0)),
            pl.BlockSpec((1, _DIN), lambda i: (0, 0)),
        ],
        out_specs=[
            pl.BlockSpec((_BLK, _DH), lambda i: (i, 0)),
            pl.BlockSpec((_BLK, _DIN), lambda i: (i, 0)),
        ],
        out_shape=[
            jax.ShapeDtypeStruct((_NP, _DH), jnp.float32),
            jax.ShapeDtypeStruct((_NP, _DIN), jnp.float32),
        ],
    )(sp, g2, degp, b2, Wd, bd)


def kernel(x, edge_index, W1, b1, W2, b2, Wd, bd):
    ei = edge_index.astype(jnp.int32)
    pad = jnp.full((_EPAD - _E,), _N, jnp.int32)
    src = jnp.concatenate([ei[0], pad]).reshape(_NW, _NCH, _C)
    dst = jnp.concatenate([ei[1], pad]).reshape(_NW, _NCH, _C)

    ones8 = jnp.ones((_C, 8), jnp.float32)
    zeros8 = jnp.zeros((_RPS, 8), jnp.float32)
    zeros64 = jnp.zeros((_RPS, _DH), jnp.float32)
    xp = jnp.concatenate([x, jnp.zeros((_NP - _N, _DIN), jnp.float32)])
    b1r = b1.reshape(1, _DH)
    b2r = b2.reshape(1, _DH)
    bdr = bd.reshape(1, _DIN)

    degp = _deg(dst, ones8, zeros8)
    g1 = _mm1(xp, W1, degp)
    sp1 = _agg(g1, src, dst, zeros64)
    g2 = _mm2(sp1, g1, degp, b1r, W2)
    sp2 = _agg(g2, src, dst, zeros64)
    h, dec = _mm3(sp2, g2, degp, b2r, Wd, bdr)
    return h[:_N], dec[:_N]


# speedup vs baseline: 16.6160x; 1.0943x over previous
import functools

import jax
import jax.numpy as jnp
from jax import lax
from jax.experimental import pallas as pl
from jax.experimental.pallas import tpu as pltpu
from jax.experimental.pallas import tpu_sc as plsc

_N = 10000
_E = 320000
_DIN = 128
_DH = 64

_NC = 2
_NS = 16
_NW = _NC * _NS
_C = 128
_NCH = 80
_EPAD = _NW * _NCH * _C
_NP = 10240
_RPS = _NP // _NS

_MESH = dict(core_axis_name="c", subcore_axis_name="s", num_cores=_NC,
             num_subcores=_NS)


@functools.partial(
    pl.kernel,
    out_type=jax.ShapeDtypeStruct((_NC, _NP, 8), jnp.float32),
    mesh=plsc.VectorSubcoreMesh(**_MESH),
    scratch_types=[
        pltpu.VMEM((_NCH, _C), jnp.int32),
        pltpu.VMEM((_C, 8), jnp.float32),
        pltpu.VMEM_SHARED((_NP, 8), jnp.float32),
    ],
    compiler_params=pltpu.CompilerParams(use_tc_tiling_on_sc=False),
)
def _deg(dst_hbm, ones_hbm, zeros_hbm, out_hbm, didx, ones_v, acc):
    c = lax.axis_index("c")
    s = lax.axis_index("s")
    wid = c * _NS + s
    pltpu.sync_copy(dst_hbm.at[wid], didx)
    pltpu.sync_copy(ones_hbm, ones_v)
    pltpu.sync_copy(zeros_hbm, acc.at[pl.ds(s * _RPS, _RPS)])
    plsc.subcore_barrier()

    def body(j, carry):
        pltpu.sync_copy(ones_v, acc.at[didx.at[j]], add=True)
        return carry

    lax.fori_loop(0, _NCH, body, 0)
    plsc.subcore_barrier()
    pltpu.sync_copy(acc.at[pl.ds(s * _RPS, _RPS)],
                    out_hbm.at[c, pl.ds(s * _RPS, _RPS)])


@functools.partial(
    pl.kernel,
    out_type=jax.ShapeDtypeStruct((_NC, _NP, _DH), jnp.float32),
    mesh=plsc.VectorSubcoreMesh(**_MESH),
    scratch_types=[
        pltpu.VMEM((_NCH, _C), jnp.int32),
        pltpu.VMEM((_NCH, _C), jnp.int32),
        [pltpu.VMEM((_C, _DH), jnp.float32) for _ in range(4)],
        [pltpu.SemaphoreType.DMA for _ in range(4)],
        [pltpu.SemaphoreType.DMA for _ in range(4)],
        pltpu.VMEM_SHARED((_NP, _DH), jnp.float32),
    ],
    compiler_params=pltpu.CompilerParams(use_tc_tiling_on_sc=False),
)
def _agg(g_hbm, src_hbm, dst_hbm, zeros_hbm, out_hbm,
         sidx, didx, rows, gsem, ssem, acc):
    c = lax.axis_index("c")
    s = lax.axis_index("s")
    wid = c * _NS + s
    pltpu.sync_copy(src_hbm.at[wid], sidx)
    pltpu.sync_copy(dst_hbm.at[wid], didx)
    pltpu.sync_copy(zeros_hbm, acc.at[pl.ds(s * _RPS, _RPS)])
    plsc.subcore_barrier()

    def gath(j, b):
        pltpu.async_copy(g_hbm.at[sidx.at[j]], rows[b], gsem[b])

    def gath_wait(j, b):
        pltpu.make_async_copy(g_hbm.at[sidx.at[j]], rows[b], gsem[b]).wait()

    def scat(j, b):
        pltpu.async_copy(rows[b], acc.at[didx.at[j]], ssem[b], add=True)

    def scat_wait(j, b):
        pltpu.make_async_copy(rows[b], acc.at[didx.at[j]], ssem[b]).wait()

    gath(0, 0)
    gath(1, 1)
    gath_wait(0, 0)
    scat(0, 0)
    gath(2, 2)
    gath_wait(1, 1)
    scat(1, 1)
    gath(3, 3)

    def body(i, carry):
        j0 = 2 + 4 * i
        for k in range(4):
            j = j0 + k
            b = (2 + k) % 4
            gath_wait(j, b)
            scat(j, b)
            bn = (b + 2) % 4
            scat_wait(j - 2, bn)
            gath(j + 2, bn)
        return carry

    lax.fori_loop(0, (_NCH - 4) // 4, body, 0)
    gath_wait(_NCH - 2, 2)
    scat(_NCH - 2, 2)
    scat_wait(_NCH - 4, 0)
    gath_wait(_NCH - 1, 3)
    scat(_NCH - 1, 3)
    scat_wait(_NCH - 3, 1)
    scat_wait(_NCH - 2, 2)
    scat_wait(_NCH - 1, 3)
    plsc.subcore_barrier()
    pltpu.sync_copy(acc.at[pl.ds(s * _RPS, _RPS)],
                    out_hbm.at[c, pl.ds(s * _RPS, _RPS)])


_BLK = 1024
_G = _NP // _BLK


def _dis_block(degp_ref):
    deg = degp_ref[0, :, 0:1] + degp_ref[1, :, 0:1] + 1.0
    return lax.rsqrt(deg)


def _mm1_body(x_ref, w_ref, degp_ref, out_ref):
    dis = _dis_block(degp_ref)
    h = jnp.dot(x_ref[...], w_ref[...], preferred_element_type=jnp.float32)
    out_ref[...] = h * dis


def _mm2_body(sp_ref, g_ref, degp_ref, b_ref, w_ref, out_ref):
    dis = _dis_block(degp_ref)
    ssum = sp_ref[0] + sp_ref[1] + g_ref[...]
    h = jnp.maximum(ssum * dis + b_ref[...], 0.0)
    out_ref[...] = jnp.dot(h, w_ref[...],
                           preferred_element_type=jnp.float32) * dis


def _mm3_body(sp_ref, g_ref, degp_ref, b_ref, wd_ref, bd_ref,
              h_ref, dec_ref):
    dis = _dis_block(degp_ref)
    ssum = sp_ref[0] + sp_ref[1] + g_ref[...]
    h = jnp.maximum(ssum * dis + b_ref[...], 0.0)
    h_ref[...] = h
    dec_ref[...] = jnp.dot(h, wd_ref[...],
                           preferred_element_type=jnp.float32) + bd_ref[...]


def _mm1(xp, W1, degp):
    return pl.pallas_call(
        _mm1_body,
        grid=(_G,),
        in_specs=[
            pl.BlockSpec((_BLK, _DIN), lambda i: (i, 0)),
            pl.BlockSpec((_DIN, _DH), lambda i: (0, 0)),
            pl.BlockSpec((_NC, _BLK, 8), lambda i: (0, i, 0)),
        ],
        out_specs=pl.BlockSpec((_BLK, _DH), lambda i: (i, 0)),
        out_shape=jax.ShapeDtypeStruct((_NP, _DH), jnp.float32),
    )(xp, W1, degp)


def _mm2(sp, g1, degp, b1, W2):
    return pl.pallas_call(
        _mm2_body,
        grid=(_G,),
        in_specs=[
            pl.BlockSpec((_NC, _BLK, _DH), lambda i: (0, i, 0)),
            pl.BlockSpec((_BLK, _DH), lambda i: (i, 0)),
            pl.BlockSpec((_NC, _BLK, 8), lambda i: (0, i, 0)),
            pl.BlockSpec((1, _DH), lambda i: (0, 0)),
            pl.BlockSpec((_DH, _DH), lambda i: (0, 0)),
        ],
        out_specs=pl.BlockSpec((_BLK, _DH), lambda i: (i, 0)),
        out_shape=jax.ShapeDtypeStruct((_NP, _DH), jnp.float32),
    )(sp, g1, degp, b1, W2)


def _mm3(sp, g2, degp, b2, Wd, bd):
    return pl.pallas_call(
        _mm3_body,
        grid=(_G,),
        in_specs=[
            pl.BlockSpec((_NC, _BLK, _DH), lambda i: (0, i, 0)),
            pl.BlockSpec((_BLK, _DH), lambda i: (i, 0)),
            pl.BlockSpec((_NC, _BLK, 8), lambda i: (0, i, 0)),
            pl.BlockSpec((1, _DH), lambda i: (0, 0)),
            pl.BlockSpec((_DH, _DIN), lambda i: (0, 0)),
            pl.BlockSpec((1, _DIN), lambda i: (0, 0)),
        ],
        out_specs=[
            pl.BlockSpec((_BLK, _DH), lambda i: (i, 0)),
            pl.BlockSpec((_BLK, _DIN), lambda i: (i, 0)),
        ],
        out_shape=[
            jax.ShapeDtypeStruct((_NP, _DH), jnp.float32),
            jax.ShapeDtypeStruct((_NP, _DIN), jnp.float32),
        ],
    )(sp, g2, degp, b2, Wd, bd)


def kernel(x, edge_index, W1, b1, W2, b2, Wd, bd):
    ei = edge_index.astype(jnp.int32)
    pad = jnp.full((_EPAD - _E,), _N, jnp.int32)
    src = jnp.concatenate([ei[0], pad]).reshape(_NW, _NCH, _C)
    dst = jnp.concatenate([ei[1], pad]).reshape(_NW, _NCH, _C)

    ones8 = jnp.ones((_C, 8), jnp.float32)
    zeros8 = jnp.zeros((_RPS, 8), jnp.float32)
    zeros64 = jnp.zeros((_RPS, _DH), jnp.float32)
    xp = jnp.concatenate([x, jnp.zeros((_NP - _N, _DIN), jnp.float32)])
    b1r = b1.reshape(1, _DH)
    b2r = b2.reshape(1, _DH)
    bdr = bd.reshape(1, _DIN)

    degp = _deg(dst, ones8, zeros8)
    g1 = _mm1(xp, W1, degp)
    sp1 = _agg(g1, src, dst, zeros64)
    g2 = _mm2(sp1, g1, degp, b1r, W2)
    sp2 = _agg(g2, src, dst, zeros64)
    h, dec = _mm3(sp2, g2, degp, b2r, Wd, bdr)
    return h[:_N], dec[:_N]


# speedup vs baseline: 37.7449x; 2.2716x over previous
import functools

import jax
import jax.numpy as jnp
from jax import lax
from jax.experimental import pallas as pl
from jax.experimental.pallas import tpu as pltpu
from jax.experimental.pallas import tpu_sc as plsc

_N = 10000
_E = 320000
_DIN = 128
_DH = 64

_NC = 2
_NS = 16
_NW = _NC * _NS
_C = 128
_NCH = 80
_EPAD = _NW * _NCH * _C
_NP = 10240
_RPS = _NP // _NS

_MESH = dict(core_axis_name="c", subcore_axis_name="s", num_cores=_NC,
             num_subcores=_NS)


@functools.partial(
    pl.kernel,
    out_type=jax.ShapeDtypeStruct((_NC, _NP, 8), jnp.float32),
    mesh=plsc.VectorSubcoreMesh(**_MESH),
    scratch_types=[
        pltpu.VMEM((_NCH, _C), jnp.int32),
        pltpu.VMEM((_C, 8), jnp.float32),
        pltpu.VMEM_SHARED((_NP, 8), jnp.float32),
    ],
    compiler_params=pltpu.CompilerParams(use_tc_tiling_on_sc=False),
)
def _deg(dst_hbm, ones_hbm, zeros_hbm, out_hbm, didx, ones_v, acc):
    c = lax.axis_index("c")
    s = lax.axis_index("s")
    wid = c * _NS + s
    pltpu.sync_copy(dst_hbm.at[wid], didx)
    pltpu.sync_copy(ones_hbm, ones_v)
    pltpu.sync_copy(zeros_hbm, acc.at[pl.ds(s * _RPS, _RPS)])
    plsc.subcore_barrier()

    def body(j, carry):
        pltpu.sync_copy(ones_v, acc.at[didx.at[j]], add=True)
        return carry

    lax.fori_loop(0, _NCH, body, 0)
    plsc.subcore_barrier()
    pltpu.sync_copy(acc.at[pl.ds(s * _RPS, _RPS)],
                    out_hbm.at[c, pl.ds(s * _RPS, _RPS)])


@functools.partial(
    pl.kernel,
    out_type=jax.ShapeDtypeStruct((_NC, _NP, _DH), jnp.float32),
    mesh=plsc.VectorSubcoreMesh(**_MESH),
    scratch_types=[
        pltpu.VMEM((_NCH, _C), jnp.int32),
        pltpu.VMEM((_NCH, _C), jnp.int32),
        [pltpu.VMEM((_C, _DH), jnp.float32) for _ in range(4)],
        [pltpu.SemaphoreType.DMA for _ in range(4)],
        [pltpu.SemaphoreType.DMA for _ in range(4)],
        pltpu.VMEM_SHARED((_NP, _DH), jnp.float32),
    ],
    compiler_params=pltpu.CompilerParams(use_tc_tiling_on_sc=False),
)
def _agg(g_hbm, src_hbm, dst_hbm, zeros_hbm, out_hbm,
         sidx, didx, rows, gsem, ssem, acc):
    c = lax.axis_index("c")
    s = lax.axis_index("s")
    wid = c * _NS + s
    pltpu.sync_copy(src_hbm.at[wid], sidx)
    pltpu.sync_copy(dst_hbm.at[wid], didx)
    pltpu.sync_copy(zeros_hbm, acc.at[pl.ds(s * _RPS, _RPS)])
    plsc.subcore_barrier()

    def gath(j, b):
        pltpu.async_copy(g_hbm.at[sidx.at[j]], rows[b], gsem[b])

    def gath_wait(j, b):
        pltpu.make_async_copy(g_hbm.at[sidx.at[j]], rows[b], gsem[b]).wait()

    def scat(j, b):
        pltpu.async_copy(rows[b], acc.at[didx.at[j]], ssem[b], add=True)

    def scat_wait(j, b):
        pltpu.make_async_copy(rows[b], acc.at[didx.at[j]], ssem[b]).wait()

    gath(0, 0)
    gath(1, 1)
    gath_wait(0, 0)
    scat(0, 0)
    gath(2, 2)
    gath_wait(1, 1)
    scat(1, 1)
    gath(3, 3)

    def body(i, carry):
        j0 = 2 + 4 * i
        for k in range(4):
            j = j0 + k
            b = (2 + k) % 4
            gath_wait(j, b)
            scat(j, b)
            bn = (b + 2) % 4
            scat_wait(j - 2, bn)
            gath(j + 2, bn)
        return carry

    lax.fori_loop(0, (_NCH - 4) // 4, body, 0)
    gath_wait(_NCH - 2, 2)
    scat(_NCH - 2, 2)
    scat_wait(_NCH - 4, 0)
    gath_wait(_NCH - 1, 3)
    scat(_NCH - 1, 3)
    scat_wait(_NCH - 3, 1)
    scat_wait(_NCH - 2, 2)
    scat_wait(_NCH - 1, 3)
    plsc.subcore_barrier()
    pltpu.sync_copy(acc.at[pl.ds(s * _RPS, _RPS)],
                    out_hbm.at[c, pl.ds(s * _RPS, _RPS)])


_BLK = 1000
_G = _N // _BLK


def _dis_block(degp_ref):
    deg = degp_ref[0, :, 0:1] + degp_ref[1, :, 0:1] + 1.0
    return lax.rsqrt(deg)


def _mm1_body(x_ref, w_ref, degp_ref, out_ref):
    dis = _dis_block(degp_ref)
    h = jnp.dot(x_ref[...], w_ref[...], preferred_element_type=jnp.float32)
    out_ref[...] = h * dis


def _mm2_body(sp_ref, g_ref, degp_ref, b_ref, w_ref, out_ref):
    dis = _dis_block(degp_ref)
    ssum = sp_ref[0] + sp_ref[1] + g_ref[...]
    h = jnp.maximum(ssum * dis + b_ref[...], 0.0)
    out_ref[...] = jnp.dot(h, w_ref[...],
                           preferred_element_type=jnp.float32) * dis


def _mm3_body(sp_ref, g_ref, degp_ref, b_ref, wd_ref, bd_ref,
              h_ref, dec_ref):
    dis = _dis_block(degp_ref)
    ssum = sp_ref[0] + sp_ref[1] + g_ref[...]
    h = jnp.maximum(ssum * dis + b_ref[...], 0.0)
    h_ref[...] = h
    dec_ref[...] = jnp.dot(h, wd_ref[...],
                           preferred_element_type=jnp.float32) + bd_ref[...]


def _mm1(x, W1, degp):
    return pl.pallas_call(
        _mm1_body,
        grid=(_G,),
        in_specs=[
            pl.BlockSpec((_BLK, _DIN), lambda i: (i, 0)),
            pl.BlockSpec((_DIN, _DH), lambda i: (0, 0)),
            pl.BlockSpec((_NC, _BLK, 8), lambda i: (0, i, 0)),
        ],
        out_specs=pl.BlockSpec((_BLK, _DH), lambda i: (i, 0)),
        out_shape=jax.ShapeDtypeStruct((_NP, _DH), jnp.float32),
    )(x, W1, degp)


def _mm2(sp, g1, degp, b1, W2):
    return pl.pallas_call(
        _mm2_body,
        grid=(_G,),
        in_specs=[
            pl.BlockSpec((_NC, _BLK, _DH), lambda i: (0, i, 0)),
            pl.BlockSpec((_BLK, _DH), lambda i: (i, 0)),
            pl.BlockSpec((_NC, _BLK, 8), lambda i: (0, i, 0)),
            pl.BlockSpec((1, _DH), lambda i: (0, 0)),
            pl.BlockSpec((_DH, _DH), lambda i: (0, 0)),
        ],
        out_specs=pl.BlockSpec((_BLK, _DH), lambda i: (i, 0)),
        out_shape=jax.ShapeDtypeStruct((_NP, _DH), jnp.float32),
    )(sp, g1, degp, b1, W2)


def _mm3(sp, g2, degp, b2, Wd, bd):
    return pl.pallas_call(
        _mm3_body,
        grid=(_G,),
        in_specs=[
            pl.BlockSpec((_NC, _BLK, _DH), lambda i: (0, i, 0)),
            pl.BlockSpec((_BLK, _DH), lambda i: (i, 0)),
            pl.BlockSpec((_NC, _BLK, 8), lambda i: (0, i, 0)),
            pl.BlockSpec((1, _DH), lambda i: (0, 0)),
            pl.BlockSpec((_DH, _DIN), lambda i: (0, 0)),
            pl.BlockSpec((1, _DIN), lambda i: (0, 0)),
        ],
        out_specs=[
            pl.BlockSpec((_BLK, _DH), lambda i: (i, 0)),
            pl.BlockSpec((_BLK, _DIN), lambda i: (i, 0)),
        ],
        out_shape=[
            jax.ShapeDtypeStruct((_N, _DH), jnp.float32),
            jax.ShapeDtypeStruct((_N, _DIN), jnp.float32),
        ],
    )(sp, g2, degp, b2, Wd, bd)


def kernel(x, edge_index, W1, b1, W2, b2, Wd, bd):
    ei = edge_index.astype(jnp.int32)
    pad = _N + jnp.arange(_EPAD - _E, dtype=jnp.int32) % (_NP - _N)
    src = jnp.concatenate([ei[0], pad]).reshape(_NW, _NCH, _C)
    dst = jnp.concatenate([ei[1], pad]).reshape(_NW, _NCH, _C)

    ones8 = jnp.ones((_C, 8), jnp.float32)
    zeros8 = jnp.zeros((_RPS, 8), jnp.float32)
    zeros64 = jnp.zeros((_RPS, _DH), jnp.float32)
    b1r = b1.reshape(1, _DH)
    b2r = b2.reshape(1, _DH)
    bdr = bd.reshape(1, _DIN)

    degp = _deg(dst, ones8, zeros8)
    g1 = _mm1(x, W1, degp)
    sp1 = _agg(g1, src, dst, zeros64)
    g2 = _mm2(sp1, g1, degp, b1r, W2)
    sp2 = _agg(g2, src, dst, zeros64)
    h, dec = _mm3(sp2, g2, degp, b2r, Wd, bdr)
    return h, dec


# speedup vs baseline: 39.9024x; 1.0572x over previous
import functools

import jax
import jax.numpy as jnp
from jax import lax
from jax.experimental import pallas as pl
from jax.experimental.pallas import tpu as pltpu
from jax.experimental.pallas import tpu_sc as plsc

_N = 10000
_E = 320000
_DIN = 128
_DH = 64

_NC = 2
_NS = 16
_NW = _NC * _NS
_C = 128
_NCH = 80
_EPAD = _NW * _NCH * _C
_NP = 10240
_RPS = _NP // _NS

_MESH = dict(core_axis_name="c", subcore_axis_name="s", num_cores=_NC,
             num_subcores=_NS)


@functools.partial(
    pl.kernel,
    out_type=jax.ShapeDtypeStruct((_NC, _NP, 8), jnp.float32),
    mesh=plsc.VectorSubcoreMesh(**_MESH),
    scratch_types=[
        pltpu.VMEM((_NCH, _C), jnp.int32),
        pltpu.VMEM((_C, 8), jnp.float32),
        [pltpu.SemaphoreType.DMA for _ in range(4)],
        pltpu.VMEM_SHARED((_NP, 8), jnp.float32),
    ],
    compiler_params=pltpu.CompilerParams(use_tc_tiling_on_sc=False),
)
def _deg(dst_hbm, ones_hbm, zeros_hbm, out_hbm, didx, ones_v, ssem, acc):
    c = lax.axis_index("c")
    s = lax.axis_index("s")
    wid = c * _NS + s
    pltpu.sync_copy(dst_hbm.at[wid], didx)
    pltpu.sync_copy(ones_hbm, ones_v)
    pltpu.sync_copy(zeros_hbm, acc.at[pl.ds(s * _RPS, _RPS)])
    plsc.subcore_barrier()

    def scat(j, b):
        pltpu.async_copy(ones_v, acc.at[didx.at[j]], ssem[b], add=True)

    def scat_wait(j, b):
        pltpu.make_async_copy(ones_v, acc.at[didx.at[j]], ssem[b]).wait()

    for j in range(4):
        scat(j, j)

    def body(i, carry):
        j0 = 4 + 4 * i
        for k in range(4):
            scat_wait(j0 + k - 4, k)
            scat(j0 + k, k)
        return carry

    lax.fori_loop(0, (_NCH - 4) // 4, body, 0)
    for j in range(_NCH - 4, _NCH):
        scat_wait(j, j % 4)
    plsc.subcore_barrier()
    pltpu.sync_copy(acc.at[pl.ds(s * _RPS, _RPS)],
                    out_hbm.at[c, pl.ds(s * _RPS, _RPS)])


@functools.partial(
    pl.kernel,
    out_type=jax.ShapeDtypeStruct((_NC, _NP, _DH), jnp.float32),
    mesh=plsc.VectorSubcoreMesh(**_MESH),
    scratch_types=[
        pltpu.VMEM((_NCH, _C), jnp.int32),
        pltpu.VMEM((_NCH, _C), jnp.int32),
        [pltpu.VMEM((_C, _DH), jnp.float32) for _ in range(6)],
        [pltpu.SemaphoreType.DMA for _ in range(6)],
        [pltpu.SemaphoreType.DMA for _ in range(6)],
        pltpu.VMEM_SHARED((_NP, _DH), jnp.float32),
    ],
    compiler_params=pltpu.CompilerParams(use_tc_tiling_on_sc=False),
)
def _agg(g_hbm, src_hbm, dst_hbm, zeros_hbm, out_hbm,
         sidx, didx, rows, gsem, ssem, acc):
    c = lax.axis_index("c")
    s = lax.axis_index("s")
    wid = c * _NS + s
    pltpu.sync_copy(src_hbm.at[wid], sidx)
    pltpu.sync_copy(dst_hbm.at[wid], didx)
    pltpu.sync_copy(zeros_hbm, acc.at[pl.ds(s * _RPS, _RPS)])
    plsc.subcore_barrier()

    def gath(j, b):
        pltpu.async_copy(g_hbm.at[sidx.at[j]], rows[b], gsem[b])

    def gath_wait(j, b):
        pltpu.make_async_copy(g_hbm.at[sidx.at[j]], rows[b], gsem[b]).wait()

    def scat(j, b):
        pltpu.async_copy(rows[b], acc.at[didx.at[j]], ssem[b], add=True)

    def scat_wait(j, b):
        pltpu.make_async_copy(rows[b], acc.at[didx.at[j]], ssem[b]).wait()

    for j in range(3):
        gath(j, j)
    for j in range(3):
        gath_wait(j, j)
        scat(j, j)
        gath(j + 3, j + 3)

    def body(i, carry):
        j0 = 3 + 6 * i
        for k in range(6):
            j = j0 + k
            b = (3 + k) % 6
            gath_wait(j, b)
            scat(j, b)
            bn = (b + 3) % 6
            scat_wait(j - 3, bn)
            gath(j + 3, bn)
        return carry

    lax.fori_loop(0, (_NCH - 8) // 6, body, 0)
    for j in range(_NCH - 5, _NCH):
        b = j % 6
        gath_wait(j, b)
        scat(j, b)
        bn = (b + 3) % 6
        scat_wait(j - 3, bn)
        if j + 3 < _NCH:
            gath(j + 3, bn)
    for j in range(_NCH - 3, _NCH):
        scat_wait(j, j % 6)
    plsc.subcore_barrier()
    pltpu.sync_copy(acc.at[pl.ds(s * _RPS, _RPS)],
                    out_hbm.at[c, pl.ds(s * _RPS, _RPS)])


_BLK = 1000
_G = _N // _BLK


def _dis_block(degp_ref):
    deg = degp_ref[0, :, 0:1] + degp_ref[1, :, 0:1] + 1.0
    return lax.rsqrt(deg)


def _mm1_body(x_ref, w_ref, degp_ref, out_ref):
    dis = _dis_block(degp_ref)
    h = jnp.dot(x_ref[...], w_ref[...], preferred_element_type=jnp.float32)
    out_ref[...] = h * dis


def _mm2_body(sp_ref, g_ref, degp_ref, b_ref, w_ref, out_ref):
    dis = _dis_block(degp_ref)
    ssum = sp_ref[0] + sp_ref[1] + g_ref[...]
    h = jnp.maximum(ssum * dis + b_ref[...], 0.0)
    out_ref[...] = jnp.dot(h, w_ref[...],
                           preferred_element_type=jnp.float32) * dis


def _mm3_body(sp_ref, g_ref, degp_ref, b_ref, wd_ref, bd_ref,
              h_ref, dec_ref):
    dis = _dis_block(degp_ref)
    ssum = sp_ref[0] + sp_ref[1] + g_ref[...]
    h = jnp.maximum(ssum * dis + b_ref[...], 0.0)
    h_ref[...] = h
    dec_ref[...] = jnp.dot(h, wd_ref[...],
                           preferred_element_type=jnp.float32) + bd_ref[...]


def _mm1(x, W1, degp):
    return pl.pallas_call(
        _mm1_body,
        grid=(_G,),
        in_specs=[
            pl.BlockSpec((_BLK, _DIN), lambda i: (i, 0)),
            pl.BlockSpec((_DIN, _DH), lambda i: (0, 0)),
            pl.BlockSpec((_NC, _BLK, 8), lambda i: (0, i, 0)),
        ],
        out_specs=pl.BlockSpec((_BLK, _DH), lambda i: (i, 0)),
        out_shape=jax.ShapeDtypeStruct((_NP, _DH), jnp.float32),
    )(x, W1, degp)


def _mm2(sp, g1, degp, b1, W2):
    return pl.pallas_call(
        _mm2_body,
        grid=(_G,),
        in_specs=[
            pl.BlockSpec((_NC, _BLK, _DH), lambda i: (0, i, 0)),
            pl.BlockSpec((_BLK, _DH), lambda i: (i, 0)),
            pl.BlockSpec((_NC, _BLK, 8), lambda i: (0, i, 0)),
            pl.BlockSpec((1, _DH), lambda i: (0, 0)),
            pl.BlockSpec((_DH, _DH), lambda i: (0, 0)),
        ],
        out_specs=pl.BlockSpec((_BLK, _DH), lambda i: (i, 0)),
        out_shape=jax.ShapeDtypeStruct((_NP, _DH), jnp.float32),
    )(sp, g1, degp, b1, W2)


def _mm3(sp, g2, degp, b2, Wd, bd):
    return pl.pallas_call(
        _mm3_body,
        grid=(_G,),
        in_specs=[
            pl.BlockSpec((_NC, _BLK, _DH), lambda i: (0, i, 0)),
            pl.BlockSpec((_BLK, _DH), lambda i: (i, 0)),
            pl.BlockSpec((_NC, _BLK, 8), lambda i: (0, i, 0)),
            pl.BlockSpec((1, _DH), lambda i: (0, 0)),
            pl.BlockSpec((_DH, _DIN), lambda i: (0, 0)),
            pl.BlockSpec((1, _DIN), lambda i: (0, 0)),
        ],
        out_specs=[
            pl.BlockSpec((_BLK, _DH), lambda i: (i, 0)),
            pl.BlockSpec((_BLK, _DIN), lambda i: (i, 0)),
        ],
        out_shape=[
            jax.ShapeDtypeStruct((_N, _DH), jnp.float32),
            jax.ShapeDtypeStruct((_N, _DIN), jnp.float32),
        ],
    )(sp, g2, degp, b2, Wd, bd)


def kernel(x, edge_index, W1, b1, W2, b2, Wd, bd):
    ei = edge_index.astype(jnp.int32)
    pad = _N + jnp.arange(_EPAD - _E, dtype=jnp.int32) % (_NP - _N)
    src = jnp.concatenate([ei[0], pad]).reshape(_NW, _NCH, _C)
    dst = jnp.concatenate([ei[1], pad]).reshape(_NW, _NCH, _C)

    ones8 = jnp.ones((_C, 8), jnp.float32)
    zeros8 = jnp.zeros((_RPS, 8), jnp.float32)
    zeros64 = jnp.zeros((_RPS, _DH), jnp.float32)
    b1r = b1.reshape(1, _DH)
    b2r = b2.reshape(1, _DH)
    bdr = bd.reshape(1, _DIN)

    degp = _deg(dst, ones8, zeros8)
    g1 = _mm1(x, W1, degp)
    sp1 = _agg(g1, src, dst, zeros64)
    g2 = _mm2(sp1, g1, degp, b1r, W2)
    sp2 = _agg(g2, src, dst, zeros64)
    h, dec = _mm3(sp2, g2, degp, b2r, Wd, bdr)
    return h, dec


# speedup vs baseline: 41.3135x; 1.0354x over previous
import functools

import jax
import jax.numpy as jnp
from jax import lax
from jax.experimental import pallas as pl
from jax.experimental.pallas import tpu as pltpu
from jax.experimental.pallas import tpu_sc as plsc

_N = 10000
_E = 320000
_DIN = 128
_DH = 64

_NC = 2
_NS = 16
_NW = _NC * _NS
_C = 128
_NCH = 80
_EPAD = _NW * _NCH * _C
_NP = 10240
_RPS = _NP // _NS

_MESH = dict(core_axis_name="c", subcore_axis_name="s", num_cores=_NC,
             num_subcores=_NS)


@functools.partial(
    pl.kernel,
    out_type=jax.ShapeDtypeStruct((_NC, _NP), jnp.float32),
    mesh=plsc.VectorSubcoreMesh(**_MESH),
    scratch_types=[
        pltpu.VMEM((_NCH, _C), jnp.int32),
        pltpu.VMEM((_C,), jnp.float32),
        [pltpu.SemaphoreType.DMA for _ in range(4)],
        pltpu.VMEM_SHARED((_NP,), jnp.float32),
    ],
    compiler_params=pltpu.CompilerParams(use_tc_tiling_on_sc=False),
)
def _deg(dst_hbm, ones_hbm, zeros_hbm, out_hbm, didx, ones_v, ssem, acc):
    c = lax.axis_index("c")
    s = lax.axis_index("s")
    wid = c * _NS + s
    pltpu.sync_copy(dst_hbm.at[wid], didx)
    pltpu.sync_copy(ones_hbm, ones_v)
    pltpu.sync_copy(zeros_hbm, acc.at[pl.ds(s * _RPS, _RPS)])
    plsc.subcore_barrier()

    def scat(j, b):
        pltpu.async_copy(ones_v, acc.at[didx.at[j]], ssem[b], add=True)

    def scat_wait(j, b):
        pltpu.make_async_copy(ones_v, acc.at[didx.at[j]], ssem[b]).wait()

    for j in range(4):
        scat(j, j)

    def body(i, carry):
        j0 = 4 + 4 * i
        for k in range(4):
            scat_wait(j0 + k - 4, k)
            scat(j0 + k, k)
        return carry

    lax.fori_loop(0, (_NCH - 4) // 4, body, 0)
    for j in range(_NCH - 4, _NCH):
        scat_wait(j, j % 4)
    plsc.subcore_barrier()
    pltpu.sync_copy(acc.at[pl.ds(s * _RPS, _RPS)],
                    out_hbm.at[c, pl.ds(s * _RPS, _RPS)])


@functools.partial(
    pl.kernel,
    out_type=jax.ShapeDtypeStruct((_NC, _NP, _DH), jnp.float32),
    mesh=plsc.VectorSubcoreMesh(**_MESH),
    scratch_types=[
        pltpu.VMEM((_NCH, _C), jnp.int32),
        pltpu.VMEM((_NCH, _C), jnp.int32),
        [pltpu.VMEM((_C, _DH), jnp.float32) for _ in range(6)],
        [pltpu.SemaphoreType.DMA for _ in range(6)],
        [pltpu.SemaphoreType.DMA for _ in range(6)],
        pltpu.VMEM_SHARED((_NP, _DH), jnp.float32),
    ],
    compiler_params=pltpu.CompilerParams(use_tc_tiling_on_sc=False),
)
def _agg(g_hbm, src_hbm, dst_hbm, zeros_hbm, out_hbm,
         sidx, didx, rows, gsem, ssem, acc):
    c = lax.axis_index("c")
    s = lax.axis_index("s")
    wid = c * _NS + s
    pltpu.sync_copy(src_hbm.at[wid], sidx)
    pltpu.sync_copy(dst_hbm.at[wid], didx)
    pltpu.sync_copy(zeros_hbm, acc.at[pl.ds(s * _RPS, _RPS)])
    plsc.subcore_barrier()

    def gath(j, b):
        pltpu.async_copy(g_hbm.at[sidx.at[j]], rows[b], gsem[b])

    def gath_wait(j, b):
        pltpu.make_async_copy(g_hbm.at[sidx.at[j]], rows[b], gsem[b]).wait()

    def scat(j, b):
        pltpu.async_copy(rows[b], acc.at[didx.at[j]], ssem[b], add=True)

    def scat_wait(j, b):
        pltpu.make_async_copy(rows[b], acc.at[didx.at[j]], ssem[b]).wait()

    for j in range(3):
        gath(j, j)
    for j in range(3):
        gath_wait(j, j)
        scat(j, j)
        gath(j + 3, j + 3)

    def body(i, carry):
        j0 = 3 + 6 * i
        for k in range(6):
            j = j0 + k
            b = (3 + k) % 6
            gath_wait(j, b)
            scat(j, b)
            bn = (b + 3) % 6
            scat_wait(j - 3, bn)
            gath(j + 3, bn)
        return carry

    lax.fori_loop(0, (_NCH - 8) // 6, body, 0)
    for j in range(_NCH - 5, _NCH):
        b = j % 6
        gath_wait(j, b)
        scat(j, b)
        bn = (b + 3) % 6
        scat_wait(j - 3, bn)
        if j + 3 < _NCH:
            gath(j + 3, bn)
    for j in range(_NCH - 3, _NCH):
        scat_wait(j, j % 6)
    plsc.subcore_barrier()
    pltpu.sync_copy(acc.at[pl.ds(s * _RPS, _RPS)],
                    out_hbm.at[c, pl.ds(s * _RPS, _RPS)])


_BLK = 1024
_G = _NP // _BLK


def _dis_block(degp_ref):
    deg = degp_ref[0:1, :] + degp_ref[1:2, :] + 1.0
    return jnp.transpose(lax.rsqrt(deg), (1, 0))


def _mm1_body(x_ref, w_ref, degp_ref, out_ref):
    dis = _dis_block(degp_ref)
    h = jnp.dot(x_ref[...], w_ref[...], preferred_element_type=jnp.float32)
    out_ref[...] = h * dis


def _mm2_body(sp_ref, g_ref, degp_ref, b_ref, w_ref, out_ref):
    dis = _dis_block(degp_ref)
    ssum = sp_ref[0] + sp_ref[1] + g_ref[...]
    h = jnp.maximum(ssum * dis + b_ref[...], 0.0)
    out_ref[...] = jnp.dot(h, w_ref[...],
                           preferred_element_type=jnp.float32) * dis


def _mm3_body(sp_ref, g_ref, degp_ref, b_ref, wd_ref, bd_ref,
              h_ref, dec_ref):
    dis = _dis_block(degp_ref)
    ssum = sp_ref[0] + sp_ref[1] + g_ref[...]
    h = jnp.maximum(ssum * dis + b_ref[...], 0.0)
    h_ref[...] = h
    dec_ref[...] = jnp.dot(h, wd_ref[...],
                           preferred_element_type=jnp.float32) + bd_ref[...]


def _mm1(x, W1, degp):
    return pl.pallas_call(
        _mm1_body,
        grid=(_G,),
        in_specs=[
            pl.BlockSpec((_BLK, _DIN), lambda i: (i, 0)),
            pl.BlockSpec((_DIN, _DH), lambda i: (0, 0)),
            pl.BlockSpec((_NC, _BLK), lambda i: (0, i)),
        ],
        out_specs=pl.BlockSpec((_BLK, _DH), lambda i: (i, 0)),
        out_shape=jax.ShapeDtypeStruct((_NP, _DH), jnp.float32),
    )(x, W1, degp)


def _mm2(sp, g1, degp, b1, W2):
    return pl.pallas_call(
        _mm2_body,
        grid=(_G,),
        in_specs=[
            pl.BlockSpec((_NC, _BLK, _DH), lambda i: (0, i, 0)),
            pl.BlockSpec((_BLK, _DH), lambda i: (i, 0)),
            pl.BlockSpec((_NC, _BLK), lambda i: (0, i)),
            pl.BlockSpec((1, _DH), lambda i: (0, 0)),
            pl.BlockSpec((_DH, _DH), lambda i: (0, 0)),
        ],
        out_specs=pl.BlockSpec((_BLK, _DH), lambda i: (i, 0)),
        out_shape=jax.ShapeDtypeStruct((_NP, _DH), jnp.float32),
    )(sp, g1, degp, b1, W2)


def _mm3(sp, g2, degp, b2, Wd, bd):
    return pl.pallas_call(
        _mm3_body,
        grid=(_G,),
        in_specs=[
            pl.BlockSpec((_NC, _BLK, _DH), lambda i: (0, i, 0)),
            pl.BlockSpec((_BLK, _DH), lambda i: (i, 0)),
            pl.BlockSpec((_NC, _BLK), lambda i: (0, i)),
            pl.BlockSpec((1, _DH), lambda i: (0, 0)),
            pl.BlockSpec((_DH, _DIN), lambda i: (0, 0)),
            pl.BlockSpec((1, _DIN), lambda i: (0, 0)),
        ],
        out_specs=[
            pl.BlockSpec((_BLK, _DH), lambda i: (i, 0)),
            pl.BlockSpec((_BLK, _DIN), lambda i: (i, 0)),
        ],
        out_shape=[
            jax.ShapeDtypeStruct((_NP, _DH), jnp.float32),
            jax.ShapeDtypeStruct((_NP, _DIN), jnp.float32),
        ],
    )(sp, g2, degp, b2, Wd, bd)


def kernel(x, edge_index, W1, b1, W2, b2, Wd, bd):
    ei = edge_index.astype(jnp.int32)
    pad = _N + jnp.arange(_EPAD - _E, dtype=jnp.int32) % (_NP - _N)
    src = jnp.concatenate([ei[0], pad]).reshape(_NW, _NCH, _C)
    dst = jnp.concatenate([ei[1], pad]).reshape(_NW, _NCH, _C)

    ones1 = jnp.ones((_C,), jnp.float32)
    zeros1 = jnp.zeros((_RPS,), jnp.float32)
    zeros64 = jnp.zeros((_RPS, _DH), jnp.float32)
    b1r = b1.reshape(1, _DH)
    b2r = b2.reshape(1, _DH)
    bdr = bd.reshape(1, _DIN)

    degp = _deg(dst, ones1, zeros1)
    g1 = _mm1(x, W1, degp)
    sp1 = _agg(g1, src, dst, zeros64)
    g2 = _mm2(sp1, g1, degp, b1r, W2)
    sp2 = _agg(g2, src, dst, zeros64)
    h, dec = _mm3(sp2, g2, degp, b2r, Wd, bdr)
    return h[:_N], dec[:_N]


# speedup vs baseline: 44.8414x; 1.0854x over previous
import functools

import jax
import jax.numpy as jnp
from jax import lax
from jax.experimental import pallas as pl
from jax.experimental.pallas import tpu as pltpu
from jax.experimental.pallas import tpu_sc as plsc

_N = 10000
_E = 320000
_DIN = 128
_DH = 64

_NC = 2
_NS = 16
_NW = _NC * _NS
_C = 128
_NCH = 80
_EPAD = _NW * _NCH * _C
_NP = 10240
_RPS = _NP // _NS

_MESH = dict(core_axis_name="c", subcore_axis_name="s", num_cores=_NC,
             num_subcores=_NS)


@functools.partial(
    pl.kernel,
    out_type=jax.ShapeDtypeStruct((_NC, _NP), jnp.float32),
    mesh=plsc.VectorSubcoreMesh(**_MESH),
    scratch_types=[
        pltpu.VMEM((_NCH, _C), jnp.int32),
        pltpu.VMEM((_C,), jnp.float32),
        [pltpu.SemaphoreType.DMA for _ in range(4)],
        pltpu.VMEM_SHARED((_NP,), jnp.float32),
    ],
    compiler_params=pltpu.CompilerParams(use_tc_tiling_on_sc=False),
)
def _deg(dst_hbm, ones_hbm, zeros_hbm, out_hbm, didx, ones_v, ssem, acc):
    c = lax.axis_index("c")
    s = lax.axis_index("s")
    wid = c * _NS + s
    pltpu.sync_copy(dst_hbm.at[wid], didx)
    pltpu.sync_copy(ones_hbm, ones_v)
    pltpu.sync_copy(zeros_hbm, acc.at[pl.ds(s * _RPS, _RPS)])
    plsc.subcore_barrier()

    def scat(j, b):
        pltpu.async_copy(ones_v, acc.at[didx.at[j]], ssem[b], add=True)

    def scat_wait(j, b):
        pltpu.make_async_copy(ones_v, acc.at[didx.at[j]], ssem[b]).wait()

    for j in range(4):
        scat(j, j)

    def body(i, carry):
        j0 = 4 + 4 * i
        for k in range(4):
            scat_wait(j0 + k - 4, k)
            scat(j0 + k, k)
        return carry

    lax.fori_loop(0, (_NCH - 4) // 4, body, 0)
    for j in range(_NCH - 4, _NCH):
        scat_wait(j, j % 4)
    plsc.subcore_barrier()
    pltpu.sync_copy(acc.at[pl.ds(s * _RPS, _RPS)],
                    out_hbm.at[c, pl.ds(s * _RPS, _RPS)])


@functools.partial(
    pl.kernel,
    out_type=jax.ShapeDtypeStruct((_NP, _NC * _DH), jnp.float32),
    mesh=plsc.VectorSubcoreMesh(**_MESH),
    scratch_types=[
        pltpu.VMEM((_NCH, _C), jnp.int32),
        pltpu.VMEM((_NCH, _C), jnp.int32),
        [pltpu.VMEM((_C, _DH), jnp.float32) for _ in range(6)],
        [pltpu.SemaphoreType.DMA for _ in range(6)],
        [pltpu.SemaphoreType.DMA for _ in range(6)],
        pltpu.VMEM_SHARED((_NP, _DH), jnp.float32),
    ],
    compiler_params=pltpu.CompilerParams(use_tc_tiling_on_sc=False),
)
def _agg(g_hbm, src_hbm, dst_hbm, zeros_hbm, out_hbm,
         sidx, didx, rows, gsem, ssem, acc):
    c = lax.axis_index("c")
    s = lax.axis_index("s")
    wid = c * _NS + s
    pltpu.sync_copy(src_hbm.at[wid], sidx)
    pltpu.sync_copy(dst_hbm.at[wid], didx)
    pltpu.sync_copy(zeros_hbm, acc.at[pl.ds(s * _RPS, _RPS)])
    plsc.subcore_barrier()

    def gath(j, b):
        pltpu.async_copy(g_hbm.at[sidx.at[j]], rows[b], gsem[b])

    def gath_wait(j, b):
        pltpu.make_async_copy(g_hbm.at[sidx.at[j]], rows[b], gsem[b]).wait()

    def scat(j, b):
        pltpu.async_copy(rows[b], acc.at[didx.at[j]], ssem[b], add=True)

    def scat_wait(j, b):
        pltpu.make_async_copy(rows[b], acc.at[didx.at[j]], ssem[b]).wait()

    for j in range(3):
        gath(j, j)
    for j in range(3):
        gath_wait(j, j)
        scat(j, j)
        gath(j + 3, j + 3)

    def body(i, carry):
        j0 = 3 + 6 * i
        for k in range(6):
            j = j0 + k
            b = (3 + k) % 6
            gath_wait(j, b)
            scat(j, b)
            bn = (b + 3) % 6
            scat_wait(j - 3, bn)
            gath(j + 3, bn)
        return carry

    lax.fori_loop(0, (_NCH - 8) // 6, body, 0)
    for j in range(_NCH - 5, _NCH):
        b = j % 6
        gath_wait(j, b)
        scat(j, b)
        bn = (b + 3) % 6
        scat_wait(j - 3, bn)
        if j + 3 < _NCH:
            gath(j + 3, bn)
    for j in range(_NCH - 3, _NCH):
        scat_wait(j, j % 6)
    plsc.subcore_barrier()
    pltpu.sync_copy(acc.at[pl.ds(s * _RPS, _RPS)],
                    out_hbm.at[pl.ds(s * _RPS, _RPS), pl.ds(c * _DH, _DH)])


_BLK = 1024
_G = _NP // _BLK


def _dis_block(degp_ref):
    deg = degp_ref[0:1, :] + degp_ref[1:2, :] + 1.0
    return jnp.transpose(lax.rsqrt(deg), (1, 0))


def _mm1_body(x_ref, w_ref, degp_ref, out_ref):
    dis = _dis_block(degp_ref)
    h = jnp.dot(x_ref[...], w_ref[...], preferred_element_type=jnp.float32)
    out_ref[...] = h * dis


def _mm2_body(sp_ref, g_ref, degp_ref, b_ref, w_ref, out_ref):
    dis = _dis_block(degp_ref)
    ssum = sp_ref[:, :_DH] + sp_ref[:, _DH:] + g_ref[...]
    h = jnp.maximum(ssum * dis + b_ref[...], 0.0)
    out_ref[...] = jnp.dot(h, w_ref[...],
                           preferred_element_type=jnp.float32) * dis


def _mm3_body(sp_ref, g_ref, degp_ref, b_ref, wd_ref, bd_ref,
              h_ref, dec_ref):
    dis = _dis_block(degp_ref)
    ssum = sp_ref[:, :_DH] + sp_ref[:, _DH:] + g_ref[...]
    h = jnp.maximum(ssum * dis + b_ref[...], 0.0)
    h_ref[...] = h
    dec_ref[...] = jnp.dot(h, wd_ref[...],
                           preferred_element_type=jnp.float32) + bd_ref[...]


def _mm1(x, W1, degp):
    return pl.pallas_call(
        _mm1_body,
        grid=(_G,),
        in_specs=[
            pl.BlockSpec((_BLK, _DIN), lambda i: (i, 0)),
            pl.BlockSpec((_DIN, _DH), lambda i: (0, 0)),
            pl.BlockSpec((_NC, _BLK), lambda i: (0, i)),
        ],
        out_specs=pl.BlockSpec((_BLK, _DH), lambda i: (i, 0)),
        out_shape=jax.ShapeDtypeStruct((_NP, _DH), jnp.float32),
    )(x, W1, degp)


def _mm2(sp, g1, degp, b1, W2):
    return pl.pallas_call(
        _mm2_body,
        grid=(_G,),
        in_specs=[
            pl.BlockSpec((_BLK, _NC * _DH), lambda i: (i, 0)),
            pl.BlockSpec((_BLK, _DH), lambda i: (i, 0)),
            pl.BlockSpec((_NC, _BLK), lambda i: (0, i)),
            pl.BlockSpec((1, _DH), lambda i: (0, 0)),
            pl.BlockSpec((_DH, _DH), lambda i: (0, 0)),
        ],
        out_specs=pl.BlockSpec((_BLK, _DH), lambda i: (i, 0)),
        out_shape=jax.ShapeDtypeStruct((_NP, _DH), jnp.float32),
    )(sp, g1, degp, b1, W2)


def _mm3(sp, g2, degp, b2, Wd, bd):
    return pl.pallas_call(
        _mm3_body,
        grid=(_G,),
        in_specs=[
            pl.BlockSpec((_BLK, _NC * _DH), lambda i: (i, 0)),
            pl.BlockSpec((_BLK, _DH), lambda i: (i, 0)),
            pl.BlockSpec((_NC, _BLK), lambda i: (0, i)),
            pl.BlockSpec((1, _DH), lambda i: (0, 0)),
            pl.BlockSpec((_DH, _DIN), lambda i: (0, 0)),
            pl.BlockSpec((1, _DIN), lambda i: (0, 0)),
        ],
        out_specs=[
            pl.BlockSpec((_BLK, _DH), lambda i: (i, 0)),
            pl.BlockSpec((_BLK, _DIN), lambda i: (i, 0)),
        ],
        out_shape=[
            jax.ShapeDtypeStruct((_NP, _DH), jnp.float32),
            jax.ShapeDtypeStruct((_NP, _DIN), jnp.float32),
        ],
    )(sp, g2, degp, b2, Wd, bd)


def kernel(x, edge_index, W1, b1, W2, b2, Wd, bd):
    ei = edge_index.astype(jnp.int32)
    pad = _N + jnp.arange(_EPAD - _E, dtype=jnp.int32) % (_NP - _N)
    src = jnp.concatenate([ei[0], pad]).reshape(_NW, _NCH, _C)
    dst = jnp.concatenate([ei[1], pad]).reshape(_NW, _NCH, _C)

    ones1 = jnp.ones((_C,), jnp.float32)
    zeros1 = jnp.zeros((_RPS,), jnp.float32)
    zeros64 = jnp.zeros((_RPS, _DH), jnp.float32)
    b1r = b1.reshape(1, _DH)
    b2r = b2.reshape(1, _DH)
    bdr = bd.reshape(1, _DIN)

    degp = _deg(dst, ones1, zeros1)
    g1 = _mm1(x, W1, degp)
    sp1 = _agg(g1, src, dst, zeros64)
    g2 = _mm2(sp1, g1, degp, b1r, W2)
    sp2 = _agg(g2, src, dst, zeros64)
    h, dec = _mm3(sp2, g2, degp, b2r, Wd, bdr)
    return h[:_N], dec[:_N]
